# baseline (speedup 1.0000x reference)
import functools

import jax
import jax.numpy as jnp
from jax import lax
from jax.experimental import pallas as pl
from jax.experimental.pallas import tpu as pltpu
from jax.experimental.pallas import tpu_sc as plsc


_BB = 64


def _argmax_body(n, x_ref, out_ref, acc_ref):
    j = pl.program_id(1)
    nj = pl.num_programs(1)
    blk = x_ref[...]
    v = blk.shape[-1]
    m = jnp.max(blk, axis=-1, keepdims=True)
    col = lax.broadcasted_iota(jnp.int32, blk.shape, 2).astype(jnp.float32)
    rev = jnp.float32(v - 1) - col
    cand = jnp.where(blk == m, rev, jnp.float32(-1))
    tok8 = (jnp.float32(v - 1) - jnp.max(cand, axis=-1)).astype(jnp.int32)
    for jj in range(nj):

        @pl.when(j == jj)
        def _():
            acc_ref[:, jj * 8 : (jj + 1) * 8] = tok8

    @pl.when(j == nj - 1)
    def _():
        out_ref[...] = acc_ref[:, :n]


def _tc_argmax(x, interpret=False):
    b, n, v = x.shape
    nb = b // _BB
    nj = (n + 7) // 8
    return pl.pallas_call(
        functools.partial(_argmax_body, n),
        grid=(nb, nj),
        in_specs=[pl.BlockSpec((_BB, 8, v), lambda i, j: (i, j, 0))],
        out_specs=pl.BlockSpec((_BB, n), lambda i, j: (i, 0)),
        out_shape=jax.ShapeDtypeStruct((b, n), jnp.int32),
        scratch_shapes=[pltpu.VMEM((_BB, 8 * nj), jnp.int32)],
        compiler_params=pltpu.CompilerParams(
            dimension_semantics=("arbitrary", "arbitrary"),
            vmem_limit_bytes=56 * 1024 * 1024,
        ),
        interpret=interpret,
    )(x)


def _sc_gather(tokens, table):
    info = plsc.get_sparse_core_info()
    nc, ns = info.num_cores, info.num_subcores
    nw = nc * ns
    b, n = tokens.shape
    d = table.shape[1]
    bpt = b // nw

    mesh = plsc.VectorSubcoreMesh(core_axis_name="c", subcore_axis_name="s")

    @functools.partial(
        pl.kernel,
        mesh=mesh,
        out_type=jax.ShapeDtypeStruct((b, n, d), jnp.float32),
        scratch_types=[
            pltpu.VMEM((bpt, n), jnp.int32),
            pltpu.VMEM((n, d), jnp.float32),
            pltpu.VMEM((n, d), jnp.float32),
            pltpu.SemaphoreType.DMA,
            pltpu.SemaphoreType.DMA,
        ],
        compiler_params=pltpu.CompilerParams(use_tc_tiling_on_sc=True),
    )
    def gather_kernel(tok_hbm, tab_hbm, out_hbm, idx_v, rows0, rows1, s0, s1):
        wid = lax.axis_index("s") * nc + lax.axis_index("c")
        b0 = wid * bpt
        pltpu.sync_copy(tok_hbm.at[pl.ds(b0, bpt), :], idx_v)
        bufs = (rows0, rows1)
        sems = (s0, s1)
        pending = pltpu.async_copy(tab_hbm.at[idx_v.at[0]], bufs[0], sems[0])
        for j in range(bpt):
            cur = bufs[j % 2]
            pending.wait()
            if j + 1 < bpt:
                pending = pltpu.async_copy(
                    tab_hbm.at[idx_v.at[j + 1]], bufs[(j + 1) % 2], sems[(j + 1) % 2]
                )
            pltpu.sync_copy(cur, out_hbm.at[b0 + j])

    return gather_kernel(tokens, table)


def kernel(x, dictionary):
    tokens = _tc_argmax(x)
    return _sc_gather(tokens, dictionary)


# speedup vs baseline: 1.1271x; 1.1271x over previous
import functools

import jax
import jax.numpy as jnp
from jax import lax
from jax.experimental import pallas as pl
from jax.experimental.pallas import tpu as pltpu
from jax.experimental.pallas import tpu_sc as plsc


_BB = 64


def _argmax_body(x_hbm, out_ref, buf, sem):
    i = pl.program_id(0)
    ni = pl.num_programs(0)
    p = lax.rem(i, 2)
    q = lax.rem(i + 1, 2)

    @pl.when(i == 0)
    def _():
        pltpu.make_async_copy(
            x_hbm.at[pl.ds(0, _BB)], buf.at[0], sem.at[0]
        ).start()

    @pl.when(i + 1 < ni)
    def _():
        pltpu.make_async_copy(
            x_hbm.at[pl.ds((i + 1) * _BB, _BB)], buf.at[q], sem.at[q]
        ).start()

    pltpu.make_async_copy(
        x_hbm.at[pl.ds(i * _BB, _BB)], buf.at[p], sem.at[p]
    ).wait()

    blk = buf[p]
    v = blk.shape[-1]
    m = jnp.max(blk, axis=-1, keepdims=True)
    col = lax.broadcasted_iota(jnp.int32, blk.shape, 2).astype(jnp.float32)
    rev = jnp.float32(v - 1) - col
    cand = jnp.where(blk == m, rev, jnp.float32(-1))
    out_ref[...] = (jnp.float32(v - 1) - jnp.max(cand, axis=-1)).astype(
        jnp.int32
    )


def _tc_argmax(x, interpret=False):
    b, n, v = x.shape
    nb = b // _BB
    return pl.pallas_call(
        _argmax_body,
        grid=(nb,),
        in_specs=[pl.BlockSpec(memory_space=pl.ANY)],
        out_specs=pl.BlockSpec((_BB, n), lambda i: (i, 0)),
        out_shape=jax.ShapeDtypeStruct((b, n), jnp.int32),
        scratch_shapes=[
            pltpu.VMEM((2, _BB, n, v), jnp.float32),
            pltpu.SemaphoreType.DMA((2,)),
        ],
        compiler_params=pltpu.CompilerParams(
            dimension_semantics=("arbitrary",),
            vmem_limit_bytes=56 * 1024 * 1024,
        ),
        interpret=interpret,
    )(x)


def _sc_gather(tokens, table):
    info = plsc.get_sparse_core_info()
    nc, ns = info.num_cores, info.num_subcores
    nw = nc * ns
    b, n = tokens.shape
    d = table.shape[1]
    bpt = b // nw

    mesh = plsc.VectorSubcoreMesh(core_axis_name="c", subcore_axis_name="s")

    @functools.partial(
        pl.kernel,
        mesh=mesh,
        out_type=jax.ShapeDtypeStruct((b, n, d), jnp.float32),
        scratch_types=[
            pltpu.VMEM((bpt, n), jnp.int32),
            pltpu.VMEM((n, d), jnp.float32),
            pltpu.VMEM((n, d), jnp.float32),
            pltpu.SemaphoreType.DMA,
            pltpu.SemaphoreType.DMA,
        ],
        compiler_params=pltpu.CompilerParams(use_tc_tiling_on_sc=True),
    )
    def gather_kernel(tok_hbm, tab_hbm, out_hbm, idx_v, rows0, rows1, s0, s1):
        wid = lax.axis_index("s") * nc + lax.axis_index("c")
        b0 = wid * bpt
        pltpu.sync_copy(tok_hbm.at[pl.ds(b0, bpt), :], idx_v)
        bufs = (rows0, rows1)
        sems = (s0, s1)
        pending = pltpu.async_copy(tab_hbm.at[idx_v.at[0]], bufs[0], sems[0])
        for j in range(bpt):
            cur = bufs[j % 2]
            pending.wait()
            if j + 1 < bpt:
                pending = pltpu.async_copy(
                    tab_hbm.at[idx_v.at[j + 1]], bufs[(j + 1) % 2], sems[(j + 1) % 2]
                )
            pltpu.sync_copy(cur, out_hbm.at[b0 + j])

    return gather_kernel(tokens, table)


def kernel(x, dictionary):
    tokens = _tc_argmax(x)
    return _sc_gather(tokens, dictionary)


# speedup vs baseline: 1.1272x; 1.0000x over previous
import functools

import jax
import jax.numpy as jnp
from jax import lax
from jax.experimental import pallas as pl
from jax.experimental.pallas import tpu as pltpu
from jax.experimental.pallas import tpu_sc as plsc


_BB = 64


def _argmax_body(x_hbm, out_ref, buf, sem):
    i = pl.program_id(0)
    ni = pl.num_programs(0)
    p = lax.rem(i, 2)
    q = lax.rem(i + 1, 2)

    ns = 4
    sb = _BB // ns

    def _start(k, b):
        for s in range(ns):
            pltpu.make_async_copy(
                x_hbm.at[pl.ds(k * _BB + s * sb, sb)],
                buf.at[b, pl.ds(s * sb, sb)],
                sem.at[b, s],
            ).start()

    def _wait(k, b):
        for s in range(ns):
            pltpu.make_async_copy(
                x_hbm.at[pl.ds(k * _BB + s * sb, sb)],
                buf.at[b, pl.ds(s * sb, sb)],
                sem.at[b, s],
            ).wait()

    @pl.when(i == 0)
    def _():
        _start(i, 0)

    @pl.when(i + 1 < ni)
    def _():
        _start(i + 1, q)

    _wait(i, p)

    blk = buf[p]
    v = blk.shape[-1]
    m = jnp.max(blk, axis=-1, keepdims=True)
    col = lax.broadcasted_iota(jnp.int32, blk.shape, 2).astype(jnp.float32)
    rev = jnp.float32(v - 1) - col
    cand = jnp.where(blk == m, rev, jnp.float32(-1))
    out_ref[...] = (jnp.float32(v - 1) - jnp.max(cand, axis=-1)).astype(
        jnp.int32
    )


def _tc_argmax(x, interpret=False):
    b, n, v = x.shape
    nb = b // _BB
    return pl.pallas_call(
        _argmax_body,
        grid=(nb,),
        in_specs=[pl.BlockSpec(memory_space=pl.ANY)],
        out_specs=pl.BlockSpec((_BB, n), lambda i: (i, 0)),
        out_shape=jax.ShapeDtypeStruct((b, n), jnp.int32),
        scratch_shapes=[
            pltpu.VMEM((2, _BB, n, v), jnp.float32),
            pltpu.SemaphoreType.DMA((2, 4)),
        ],
        compiler_params=pltpu.CompilerParams(
            dimension_semantics=("arbitrary",),
            vmem_limit_bytes=56 * 1024 * 1024,
        ),
        interpret=interpret,
    )(x)


def _sc_gather(tokens, table):
    info = plsc.get_sparse_core_info()
    nc, ns = info.num_cores, info.num_subcores
    nw = nc * ns
    b, n = tokens.shape
    d = table.shape[1]
    bpt = b // nw

    mesh = plsc.VectorSubcoreMesh(core_axis_name="c", subcore_axis_name="s")

    @functools.partial(
        pl.kernel,
        mesh=mesh,
        out_type=jax.ShapeDtypeStruct((b, n, d), jnp.float32),
        scratch_types=[
            pltpu.VMEM((bpt, n), jnp.int32),
            pltpu.VMEM((n, d), jnp.float32),
            pltpu.VMEM((n, d), jnp.float32),
            pltpu.SemaphoreType.DMA,
            pltpu.SemaphoreType.DMA,
        ],
        compiler_params=pltpu.CompilerParams(use_tc_tiling_on_sc=True),
    )
    def gather_kernel(tok_hbm, tab_hbm, out_hbm, idx_v, rows0, rows1, s0, s1):
        wid = lax.axis_index("s") * nc + lax.axis_index("c")
        b0 = wid * bpt
        pltpu.sync_copy(tok_hbm.at[pl.ds(b0, bpt), :], idx_v)
        bufs = (rows0, rows1)
        sems = (s0, s1)
        pending = pltpu.async_copy(tab_hbm.at[idx_v.at[0]], bufs[0], sems[0])
        for j in range(bpt):
            cur = bufs[j % 2]
            pending.wait()
            if j + 1 < bpt:
                pending = pltpu.async_copy(
                    tab_hbm.at[idx_v.at[j + 1]], bufs[(j + 1) % 2], sems[(j + 1) % 2]
                )
            pltpu.sync_copy(cur, out_hbm.at[b0 + j])

    return gather_kernel(tokens, table)


def kernel(x, dictionary):
    tokens = _tc_argmax(x)
    return _sc_gather(tokens, dictionary)


# speedup vs baseline: 2.8764x; 2.5519x over previous
import functools

import jax
import jax.numpy as jnp
from jax import lax
from jax.experimental import pallas as pl
from jax.experimental.pallas import tpu as pltpu
from jax.experimental.pallas import tpu_sc as plsc


_NN = 2


def _argmax_body(x_ref, out_ref):
    blk = x_ref[...]
    v = blk.shape[1]
    m = jnp.max(blk, axis=1, keepdims=True)
    col = lax.broadcasted_iota(jnp.int32, blk.shape, 1).astype(jnp.float32)
    rev = jnp.float32(v - 1) - col
    cand = jnp.where(blk == m, rev, jnp.float32(-1))
    tok = (jnp.float32(v - 1) - jnp.max(cand, axis=1)).astype(jnp.int32)
    out_ref[...] = tok.reshape(tok.shape[0], 1, tok.shape[1])


def _tc_argmax(xt, interpret=False):
    n, v, b = xt.shape
    nb = n // _NN
    out3 = pl.pallas_call(
        _argmax_body,
        grid=(nb,),
        in_specs=[pl.BlockSpec((_NN, v, b), lambda i: (i, 0, 0))],
        out_specs=pl.BlockSpec((_NN, 1, b), lambda i: (i, 0, 0)),
        out_shape=jax.ShapeDtypeStruct((n, 1, b), jnp.int32),
        compiler_params=pltpu.CompilerParams(
            dimension_semantics=("arbitrary",),
            vmem_limit_bytes=56 * 1024 * 1024,
        ),
        interpret=interpret,
    )(xt)
    return out3.reshape(n, b)


def _sc_gather(tokens, table):
    info = plsc.get_sparse_core_info()
    nc, ns = info.num_cores, info.num_subcores
    nw = nc * ns
    b, n = tokens.shape
    d = table.shape[1]
    bpt = b // nw

    mesh = plsc.VectorSubcoreMesh(core_axis_name="c", subcore_axis_name="s")

    @functools.partial(
        pl.kernel,
        mesh=mesh,
        out_type=jax.ShapeDtypeStruct((b, n, d), jnp.float32),
        scratch_types=[
            pltpu.VMEM((bpt, n), jnp.int32),
            pltpu.VMEM((n, d), jnp.float32),
            pltpu.VMEM((n, d), jnp.float32),
            pltpu.SemaphoreType.DMA,
            pltpu.SemaphoreType.DMA,
        ],
        compiler_params=pltpu.CompilerParams(use_tc_tiling_on_sc=True),
    )
    def gather_kernel(tok_hbm, tab_hbm, out_hbm, idx_v, rows0, rows1, s0, s1):
        wid = lax.axis_index("s") * nc + lax.axis_index("c")
        b0 = wid * bpt
        pltpu.sync_copy(tok_hbm.at[pl.ds(b0, bpt), :], idx_v)
        bufs = (rows0, rows1)
        sems = (s0, s1)
        pending = pltpu.async_copy(tab_hbm.at[idx_v.at[0]], bufs[0], sems[0])
        for j in range(bpt):
            cur = bufs[j % 2]
            pending.wait()
            if j + 1 < bpt:
                pending = pltpu.async_copy(
                    tab_hbm.at[idx_v.at[j + 1]],
                    bufs[(j + 1) % 2],
                    sems[(j + 1) % 2],
                )
            pltpu.sync_copy(cur, out_hbm.at[b0 + j])

    return gather_kernel(tokens, table)


def kernel(x, dictionary):
    xt = jnp.transpose(x, (1, 2, 0))
    tokens_t = _tc_argmax(xt)
    tokens = jnp.transpose(tokens_t)
    return _sc_gather(tokens, dictionary)


# speedup vs baseline: 3.0914x; 1.0748x over previous
import functools

import jax
import jax.numpy as jnp
from jax import lax
from jax.experimental import pallas as pl
from jax.experimental.pallas import tpu as pltpu
from jax.experimental.pallas import tpu_sc as plsc


_NN = 2


def _argmax_body(x_ref, out_ref):
    blk = x_ref[...]
    v = blk.shape[1]
    m = jnp.max(blk, axis=1, keepdims=True)
    col = lax.broadcasted_iota(jnp.int32, blk.shape, 1).astype(jnp.float32)
    rev = jnp.float32(v - 1) - col
    cand = jnp.where(blk == m, rev, jnp.float32(-1))
    tok = (jnp.float32(v - 1) - jnp.max(cand, axis=1)).astype(jnp.int32)
    out_ref[...] = tok.reshape(tok.shape[0], 1, tok.shape[1])


def _tc_argmax(xt, interpret=False):
    n, v, b = xt.shape
    nb = n // _NN
    out3 = pl.pallas_call(
        _argmax_body,
        grid=(nb,),
        in_specs=[pl.BlockSpec((_NN, v, b), lambda i: (i, 0, 0))],
        out_specs=pl.BlockSpec((_NN, 1, b), lambda i: (i, 0, 0)),
        out_shape=jax.ShapeDtypeStruct((n, 1, b), jnp.int32),
        compiler_params=pltpu.CompilerParams(
            dimension_semantics=("arbitrary",),
            vmem_limit_bytes=56 * 1024 * 1024,
        ),
        interpret=interpret,
    )(xt)
    return out3.reshape(n, b)


def _sc_gather_t(tokens_t, table):
    info = plsc.get_sparse_core_info()
    nc, ns = info.num_cores, info.num_subcores
    nw = nc * ns
    n, b = tokens_t.shape
    d = table.shape[1]
    bpt = b // nw

    mesh = plsc.VectorSubcoreMesh(core_axis_name="c", subcore_axis_name="s")

    @functools.partial(
        pl.kernel,
        mesh=mesh,
        out_type=jax.ShapeDtypeStruct((n, b, d), jnp.float32),
        scratch_types=[
            pltpu.VMEM((n, 128), jnp.int32),
            pltpu.VMEM((bpt, d), jnp.float32),
            pltpu.VMEM((bpt, d), jnp.float32),
            pltpu.SemaphoreType.DMA,
            pltpu.SemaphoreType.DMA,
        ],
        compiler_params=pltpu.CompilerParams(use_tc_tiling_on_sc=True),
    )
    def gather_kernel(tok_hbm, tab_hbm, out_hbm, idx_v, rows0, rows1, s0, s1):
        wid = lax.axis_index("s") * nc + lax.axis_index("c")
        b0 = wid * bpt
        g0 = (b0 // 128) * 128
        off = b0 - g0
        pltpu.sync_copy(tok_hbm.at[:, pl.ds(g0, 128)], idx_v)
        bufs = (rows0, rows1)
        sems = (s0, s1)
        pending = pltpu.async_copy(
            tab_hbm.at[idx_v.at[0, pl.ds(off, bpt)]], bufs[0], sems[0]
        )
        for j in range(n):
            cur = bufs[j % 2]
            pending.wait()
            if j + 1 < n:
                pending = pltpu.async_copy(
                    tab_hbm.at[idx_v.at[j + 1, pl.ds(off, bpt)]],
                    bufs[(j + 1) % 2],
                    sems[(j + 1) % 2],
                )
            pltpu.sync_copy(cur, out_hbm.at[j, pl.ds(b0, bpt)])

    return gather_kernel(tokens_t, table)


def kernel(x, dictionary):
    xt = jnp.transpose(x, (1, 2, 0))
    tokens_t = _tc_argmax(xt)
    out_t = _sc_gather_t(tokens_t, dictionary)
    return jnp.transpose(out_t, (1, 0, 2))
